# baseline (speedup 1.0000x reference)
import functools

import jax
import jax.numpy as jnp
from jax import lax
from jax.experimental import pallas as pl
from jax.experimental.pallas import tpu as pltpu
from jax.experimental.pallas import tpu_sc as plsc


@functools.lru_cache(maxsize=None)
def _make_sc_gather(batch, seq, vocab, dim):
    info = plsc.get_sparse_core_info()
    num_cores = info.num_cores
    num_workers = info.num_cores * info.num_subcores
    tiles_s = seq // 128
    assert (batch // 8) * tiles_s == num_workers
    mesh = plsc.VectorSubcoreMesh(core_axis_name="c", subcore_axis_name="s")

    @functools.partial(
        pl.kernel,
        mesh=mesh,
        out_type=jax.ShapeDtypeStruct((batch, seq, dim), jnp.float32),
        scratch_types=[
            pltpu.VMEM((8, 128), jnp.int32),
            pltpu.VMEM((2, 16, dim), jnp.float32),
            pltpu.SemaphoreType.DMA,
            pltpu.SemaphoreType.DMA,
            pltpu.SemaphoreType.DMA,
        ],
        compiler_params=pltpu.CompilerParams(use_tc_tiling_on_sc=True,
                                             needs_layout_passes=False),
    )
    def sc_gather(ids_hbm, tab3_hbm, out_hbm, idx_v, rows_v, gsem, ssem0,
                  ssem1):
        wid = lax.axis_index("s") * num_cores + lax.axis_index("c")
        b0 = 8 * (wid // tiles_s)
        s0 = 128 * (wid % tiles_s)
        pltpu.sync_copy(ids_hbm.at[pl.ds(b0, 8), pl.ds(s0, 128)], idx_v)
        lanes16 = lax.iota(jnp.int32, 16)

        def drain_gathers(buf):
            def d(k, c):
                pltpu.make_async_copy(
                    tab3_hbm.at[0, 0], rows_v.at[buf, k], gsem).wait()
                return c

            lax.fori_loop(0, 16, d, 0)

        def drain_store(buf, sem):
            pltpu.make_async_copy(
                rows_v.at[buf], out_hbm.at[0].at[pl.ds(0, 16), :],
                sem).wait()

        for r in range(8):
            for half in range(2):
                def group(q2, carry, r=r, half=half):
                    q = 2 * q2 + half
                    buf = half
                    sem = ssem0 if half == 0 else ssem1
                    v16 = idx_v[r, pl.ds(16 * q, 16)]

                    for k in range(16):
                        i = v16[k]
                        pltpu.async_copy(tab3_hbm.at[i >> 3, i & 7],
                                         rows_v.at[buf, k], gsem)
                    @pl.when(jnp.logical_or(q2 >= 1, r > 0))
                    def _():
                        drain_store(buf, sem)

                    drain_gathers(buf)
                    pltpu.async_copy(
                        rows_v.at[buf],
                        out_hbm.at[b0 + r].at[pl.ds(s0 + 16 * q, 16), :],
                        sem)
                    return carry

                lax.fori_loop(0, 4, group, 0)
        drain_store(0, ssem0)
        drain_store(1, ssem1)
        return None

    return sc_gather


def kernel(input_ids, embedding_table):
    batch, seq = input_ids.shape
    vocab, dim = embedding_table.shape
    ids = input_ids.astype(jnp.int32)
    tab3 = embedding_table.reshape(vocab // 8, 8, dim)
    output = _make_sc_gather(batch, seq, vocab, dim)(ids, tab3)
    return (output, embedding_table)


# speedup vs baseline: 1.0132x; 1.0132x over previous
import functools

import jax
import jax.numpy as jnp
from jax import lax
from jax.experimental import pallas as pl
from jax.experimental.pallas import tpu as pltpu
from jax.experimental.pallas import tpu_sc as plsc


@functools.lru_cache(maxsize=None)
def _make_sc_gather(batch, seq, vocab, dim):
    info = plsc.get_sparse_core_info()
    num_cores = info.num_cores
    num_workers = info.num_cores * info.num_subcores
    tiles_s = seq // 128
    assert (batch // 8) * tiles_s == num_workers
    mesh = plsc.VectorSubcoreMesh(core_axis_name="c", subcore_axis_name="s")

    @functools.partial(
        pl.kernel,
        mesh=mesh,
        out_type=jax.ShapeDtypeStruct((batch, seq, dim), jnp.float32),
        scratch_types=[
            pltpu.VMEM((8, 128), jnp.int32),
            pltpu.VMEM((2, 16, dim), jnp.float32),
            pltpu.SemaphoreType.DMA,
            pltpu.SemaphoreType.DMA,
            pltpu.SemaphoreType.DMA,
        ],
        compiler_params=pltpu.CompilerParams(use_tc_tiling_on_sc=True,
                                             needs_layout_passes=False),
    )
    def sc_gather(ids_hbm, tab3_hbm, out_hbm, idx_v, rows_v, gsem, ssem0,
                  ssem1):
        wid = lax.axis_index("s") * num_cores + lax.axis_index("c")
        b0 = 8 * (wid // tiles_s)
        s0 = 128 * (wid % tiles_s)
        pltpu.sync_copy(ids_hbm.at[pl.ds(b0, 8), pl.ds(s0, 128)], idx_v)
        lanes16 = lax.iota(jnp.int32, 16)

        def drain_gathers(buf):
            def d(k, c):
                pltpu.make_async_copy(
                    tab3_hbm.at[0, 0], rows_v.at[buf, k], gsem).wait()
                return c

            lax.fori_loop(0, 16, d, 0)

        def drain_store(buf, sem):
            pltpu.make_async_copy(
                rows_v.at[buf], out_hbm.at[0].at[pl.ds(0, 16), :],
                sem).wait()

        for r in range(8):
            for half in range(2):
                def group(q2, carry, r=r, half=half):
                    q = 2 * q2 + half
                    buf = half
                    sem = ssem0 if half == 0 else ssem1
                    v16 = idx_v[r, pl.ds(16 * q, 16)]

                    def fetch(k, c):
                        i = lax.reduce_max(
                            v16.at[lanes16 * 0 + k].get(
                                mode="promise_in_bounds"), (0,))
                        pltpu.async_copy(tab3_hbm.at[i >> 3, i & 7],
                                         rows_v.at[buf, k], gsem)
                        return c

                    lax.fori_loop(0, 16, fetch, 0)
                    @pl.when(jnp.logical_or(q2 >= 1, r > 0))
                    def _():
                        drain_store(buf, sem)

                    drain_gathers(buf)
                    pltpu.async_copy(
                        rows_v.at[buf],
                        out_hbm.at[b0 + r].at[pl.ds(s0 + 16 * q, 16), :],
                        sem)
                    return carry

                lax.fori_loop(0, 4, group, 0)
        drain_store(0, ssem0)
        drain_store(1, ssem1)
        return None

    return sc_gather


def kernel(input_ids, embedding_table):
    batch, seq = input_ids.shape
    vocab, dim = embedding_table.shape
    ids = input_ids.astype(jnp.int32)
    tab3 = embedding_table.reshape(vocab // 8, 8, dim)
    output = _make_sc_gather(batch, seq, vocab, dim)(ids, tab3)
    return (output, embedding_table)


# speedup vs baseline: 1.0344x; 1.0210x over previous
import functools

import jax
import jax.numpy as jnp
from jax import lax
from jax.experimental import pallas as pl
from jax.experimental.pallas import tpu as pltpu
from jax.experimental.pallas import tpu_sc as plsc


@functools.lru_cache(maxsize=None)
def _make_sc_gather(batch, seq, vocab, dim):
    info = plsc.get_sparse_core_info()
    num_cores = info.num_cores
    num_workers = info.num_cores * info.num_subcores
    tiles_s = seq // 128
    assert (batch // 8) * tiles_s == num_workers
    mesh = plsc.VectorSubcoreMesh(core_axis_name="c", subcore_axis_name="s")

    @functools.partial(
        pl.kernel,
        mesh=mesh,
        out_type=jax.ShapeDtypeStruct((batch, seq, dim), jnp.float32),
        scratch_types=[
            pltpu.VMEM((8, 128), jnp.int32),
            pltpu.VMEM((2, 16, dim), jnp.float32),
            pltpu.SemaphoreType.DMA,
            pltpu.SemaphoreType.DMA,
            pltpu.SemaphoreType.DMA,
            pltpu.SemaphoreType.DMA,
        ],
        compiler_params=pltpu.CompilerParams(use_tc_tiling_on_sc=True,
                                             needs_layout_passes=False),
    )
    def sc_gather(ids_hbm, tab3_hbm, out_hbm, idx_v, rows_v, gsem0, gsem1,
                  ssem0, ssem1):
        wid = lax.axis_index("s") * num_cores + lax.axis_index("c")
        b0 = 8 * (wid // tiles_s)
        s0 = 128 * (wid % tiles_s)
        pltpu.sync_copy(ids_hbm.at[pl.ds(b0, 8), pl.ds(s0, 128)], idx_v)
        lanes16 = lax.iota(jnp.int32, 16)
        gsems = (gsem0, gsem1)
        ssems = (ssem0, ssem1)

        def fetch16(r, q, buf):
            v16 = idx_v[r, pl.ds(16 * q, 16)]

            def fetch(k, c):
                i = lax.reduce_max(
                    v16.at[lanes16 * 0 + k].get(
                        mode="promise_in_bounds"), (0,))
                pltpu.async_copy(tab3_hbm.at[i >> 3, i & 7],
                                 rows_v.at[buf, k], gsems[buf])
                return c

            lax.fori_loop(0, 16, fetch, 0)

        def drain_gathers(buf):
            def d(k, c):
                pltpu.make_async_copy(
                    tab3_hbm.at[0, 0], rows_v.at[buf, k], gsems[buf]).wait()
                return c

            lax.fori_loop(0, 16, d, 0)

        def drain_store(buf):
            pltpu.make_async_copy(
                rows_v.at[buf], out_hbm.at[0].at[pl.ds(0, 16), :],
                ssems[buf]).wait()

        groups = [(r, q) for r in range(8) for q in range(8)]
        fetch16(0, 0, 0)
        for t, (r, q) in enumerate(groups):
            buf = t % 2
            if t + 1 < len(groups):
                nbuf = (t + 1) % 2
                if t >= 1:
                    drain_store(nbuf)
                nr, nq = groups[t + 1]
                fetch16(nr, nq, nbuf)
            drain_gathers(buf)
            pltpu.async_copy(
                rows_v.at[buf],
                out_hbm.at[b0 + r].at[pl.ds(s0 + 16 * q, 16), :],
                ssems[buf])
        drain_store(0)
        drain_store(1)
        return None

    return sc_gather


def kernel(input_ids, embedding_table):
    batch, seq = input_ids.shape
    vocab, dim = embedding_table.shape
    ids = input_ids.astype(jnp.int32)
    tab3 = embedding_table.reshape(vocab // 8, 8, dim)
    output = _make_sc_gather(batch, seq, vocab, dim)(ids, tab3)
    return (output, embedding_table)
